# baseline (speedup 1.0000x reference)
import functools

import jax
import jax.numpy as jnp
from jax import lax
from jax.experimental import pallas as pl
from jax.experimental.pallas import tpu as pltpu
from jax.experimental.pallas import tpu_sc as plsc

B = 16384
WINDOW = 5
NEG = 5
MWL = 5
SIZE = 64
NCTX = 2 * WINDOW
NCHR = 2 * WINDOW * MWL
WCOL = 2 * NCTX + 2
CCOL = NCHR + 1

NC = 2
NS = 16
NW = NC * NS
RPW = B // NW
C = 32
NCHUNK = RPW // C
NQ = C // 16
NGRP = RPW // 16


def _sc_body(wd, cd, emb0, emb1, emb0c,
             pos_out, neg_out,
             wdv, cdv, widx_v, cidx_v, nidx_v, tidx_v, lens_v, cn_v,
             wsum0, csum0, nrows0, trows0,
             wsum1, csum1, nrows1, trows1,
             posc, negc, sem0, sem1, semo):
    wid = lax.axis_index("s") * NC + lax.axis_index("c")
    base = wid * RPW

    h1 = pltpu.async_copy(wd.at[pl.ds(base, RPW), :], wdv, semo)
    h2 = pltpu.async_copy(cd.at[pl.ds(base, RPW), :], cdv, semo)
    h1.wait()
    h2.wait()

    lane = lax.broadcasted_iota(jnp.int32, (16,), 0)
    l5 = lane * NEG
    idx15 = jnp.full((16,), 15, jnp.int32)
    zero16 = jnp.zeros((16,), jnp.float32)

    def wcol(gq, j):
        return plsc.load_gather(
            wdv, [lane + gq * 16, jnp.full((16,), j, jnp.int32)])

    def ccol(gq, j):
        return plsc.load_gather(
            cdv, [lane + gq * 16, jnp.full((16,), j, jnp.int32)])

    def unpack(gq, carry):
        sl = pl.ds(gq * 16, 16)
        for j in range(NCTX):
            widx_v[j, sl] = wcol(gq, j)
        lens_v[sl] = 0.5 / wcol(gq, NCTX).astype(jnp.float32)
        tidx_v[sl] = wcol(gq, NCTX + 1)
        for n in range(NEG):
            plsc.store_scatter(nidx_v, [l5 + (gq * 16 * NEG + n)],
                               wcol(gq, NCTX + 2 + n))
        for j in range(NCHR):
            cidx_v[j, sl] = ccol(gq, j)
        cn_v[sl] = 0.5 / ccol(gq, NCHR).astype(jnp.float32)
        return carry

    lax.fori_loop(0, NGRP, unpack, 0)

    def lane_total(v):
        return plsc.cumsum(v).at[idx15].get(mode="promise_in_bounds")

    def zero_bufs(wsum, csum):
        def zrow(i, cz):
            for d in range(4):
                wsum[i, pl.ds(d * 16, 16)] = zero16
                csum[i, pl.ds(d * 16, 16)] = zero16
            return cz
        lax.fori_loop(0, C, zrow, 0)

    def transfers(g, wsum, csum, nrows, trows, sem):
        c0 = g * C
        ts = []
        for j in range(NCTX):
            ts.append((emb0.at[widx_v.at[j, pl.ds(c0, C)]], wsum, sem, True))
        for j in range(NCHR):
            ts.append((emb0c.at[cidx_v.at[j, pl.ds(c0, C)]], csum, sem, True))
        ts.append((emb1.at[nidx_v.at[pl.ds(c0 * NEG, 128)]],
                   nrows.at[pl.ds(0, 128)], sem, False))
        ts.append((emb1.at[nidx_v.at[pl.ds(c0 * NEG + 128, 32)]],
                   nrows.at[pl.ds(128, 32)], sem, False))
        ts.append((emb1.at[tidx_v.at[pl.ds(c0, C)]], trows, sem, False))
        return ts

    def fire(g, wsum, csum, nrows, trows, sem):
        for src, dst, s, add in transfers(g, wsum, csum, nrows, trows, sem):
            pltpu.async_copy(src, dst, s, add=add)

    def drain(g, wsum, csum, nrows, trows, sem):
        for src, dst, s, add in transfers(g, wsum, csum, nrows, trows, sem):
            pltpu.make_async_copy(src, dst, s).wait()

    def compute(g, wsum, csum, nrows, trows):
        c0 = g * C
        for q in range(NQ):
            gq = g * NQ + q
            hlv = lens_v[pl.ds(c0 + q * 16, 16)]
            hcv = cn_v[pl.ds(c0 + q * 16, 16)]

            def row(r2, carry2):
                accs = carry2
                r = q * 16 + r2
                ridx = jnp.full((16,), r2, jnp.int32)
                hl = hlv.at[ridx].get(mode="promise_in_bounds")
                hc = hcv.at[ridx].get(mode="promise_in_bounds")
                avg = []
                for d in range(4):
                    sl = pl.ds(d * 16, 16)
                    avg.append(wsum[r, sl] * hl + csum[r, sl] * hc)
                msk_r = lane == r2
                pv = avg[0] * trows[r, pl.ds(0, 16)]
                for d in range(1, 4):
                    pv = pv + avg[d] * trows[r, pl.ds(d * 16, 16)]
                out = [jnp.where(msk_r, lane_total(pv), accs[0])]
                for n in range(NEG):
                    nv = avg[0] * nrows[r * NEG + n, pl.ds(0, 16)]
                    for d in range(1, 4):
                        nv = nv + avg[d] * nrows[r * NEG + n, pl.ds(d * 16, 16)]
                    out.append(jnp.where(msk_r, lane_total(nv), accs[n + 1]))
                return tuple(out)

            accs = lax.fori_loop(0, 16, row, (zero16,) * (1 + NEG))
            posc[pl.ds(c0 + q * 16, 16)] = jnp.clip(accs[0], -10.0, 10.0)
            for n in range(NEG):
                maskv = wcol(gq, NCTX + 2 + NEG + n)
                sv = jnp.where(maskv == 1,
                               jnp.clip(accs[n + 1], -10.0, 10.0), -90.0)
                plsc.store_scatter(negc, [l5 + ((c0 + q * 16) * NEG + n)], sv)

    bufs0 = (wsum0, csum0, nrows0, trows0, sem0)
    bufs1 = (wsum1, csum1, nrows1, trows1, sem1)

    zero_bufs(wsum0, csum0)
    zero_bufs(wsum1, csum1)
    fire(0, *bufs0)
    fire(1, *bufs1)

    def body(k, carry):
        g0 = 2 * k
        for g, bufs in ((g0, bufs0), (g0 + 1, bufs1)):
            drain(g, *bufs)
            compute(g, *bufs[:4])
            zero_bufs(bufs[0], bufs[1])

            @pl.when(g + 2 < NCHUNK)
            def _():
                fire(g + 2, *bufs)
        return carry

    lax.fori_loop(0, NCHUNK // 2, body, 0)

    o1 = pltpu.async_copy(posc, pos_out.at[pl.ds(base, RPW)], semo)
    o2 = pltpu.async_copy(negc, neg_out.at[pl.ds(base * NEG, RPW * NEG)], semo)
    o1.wait()
    o2.wait()


def _sc_ips(wd, cd, emb0, emb1, emb0c):
    mesh = plsc.VectorSubcoreMesh(core_axis_name="c", subcore_axis_name="s")
    return pl.kernel(
        _sc_body,
        mesh=mesh,
        compiler_params=pltpu.CompilerParams(
            needs_layout_passes=False, use_tc_tiling_on_sc=False),
        out_type=[
            jax.ShapeDtypeStruct((B,), jnp.float32),
            jax.ShapeDtypeStruct((B * NEG,), jnp.float32),
        ],
        scratch_types=[
            pltpu.VMEM((RPW, WCOL), jnp.int32),
            pltpu.VMEM((RPW, CCOL), jnp.int32),
            pltpu.VMEM((NCTX, RPW), jnp.int32),
            pltpu.VMEM((NCHR, RPW), jnp.int32),
            pltpu.VMEM((RPW * NEG,), jnp.int32),
            pltpu.VMEM((RPW,), jnp.int32),
            pltpu.VMEM((RPW,), jnp.float32),
            pltpu.VMEM((RPW,), jnp.float32),
            pltpu.VMEM((C, SIZE), jnp.float32),
            pltpu.VMEM((C, SIZE), jnp.float32),
            pltpu.VMEM((C * NEG, SIZE), jnp.float32),
            pltpu.VMEM((C, SIZE), jnp.float32),
            pltpu.VMEM((C, SIZE), jnp.float32),
            pltpu.VMEM((C, SIZE), jnp.float32),
            pltpu.VMEM((C * NEG, SIZE), jnp.float32),
            pltpu.VMEM((C, SIZE), jnp.float32),
            pltpu.VMEM((RPW,), jnp.float32),
            pltpu.VMEM((RPW * NEG,), jnp.float32),
            pltpu.SemaphoreType.DMA,
            pltpu.SemaphoreType.DMA,
            pltpu.SemaphoreType.DMA,
        ],
    )(wd, cd, emb0, emb1, emb0c)


def _tc_loss_body(pos_ref, neg_ref, out_ref):
    pos_loss = jnp.sum(jnp.log(1.0 + jnp.exp(-pos_ref[...])))
    neg_loss = jnp.sum(jnp.log(1.0 + jnp.exp(neg_ref[...])))
    out_ref[0, 0] = pos_loss + neg_loss


def _tc_loss(pos2d, neg2d):
    return pl.pallas_call(
        _tc_loss_body,
        out_shape=jax.ShapeDtypeStruct((1, 1), jnp.float32),
        out_specs=pl.BlockSpec(memory_space=pltpu.SMEM),
    )(pos2d, neg2d)


def kernel(word_data, char_data, emb0_w, emb1_w, emb0_char_w):
    wd = word_data.astype(jnp.int32)
    cd = char_data.astype(jnp.int32)
    pos_ips, neg_ips = _sc_ips(wd, cd, emb0_w, emb1_w, emb0_char_w)
    loss = _tc_loss(pos_ips.reshape(128, 128),
                    neg_ips.reshape(B * NEG // 128, 128))
    return loss[0, 0]


# speedup vs baseline: 1.0270x; 1.0270x over previous
import functools

import jax
import jax.numpy as jnp
from jax import lax
from jax.experimental import pallas as pl
from jax.experimental.pallas import tpu as pltpu
from jax.experimental.pallas import tpu_sc as plsc

B = 16384
WINDOW = 5
NEG = 5
MWL = 5
SIZE = 64
NCTX = 2 * WINDOW
NCHR = 2 * WINDOW * MWL
WCOL = 2 * NCTX + 2
CCOL = NCHR + 1

NC = 2
NS = 16
NW = NC * NS
RPW = B // NW
C = 32
NCHUNK = RPW // C
NQ = C // 16
NGRP = RPW // 16


def _sc_body(wd, cd, emb0, emb1, emb0c,
             pos_out, neg_out,
             wdv, cdv, widx_v, cidx_v, nidx_v, mask_v, tidx_v, lens_v, cn_v,
             wsum0, csum0, nrows0, trows0,
             wsum1, csum1, nrows1, trows1,
             posc, negc, sem0, sem1, semo):
    wid = lax.axis_index("s") * NC + lax.axis_index("c")
    base = wid * RPW

    lane = lax.broadcasted_iota(jnp.int32, (16,), 0)
    l5 = lane * NEG
    idx15 = jnp.full((16,), 15, jnp.int32)
    zero16 = jnp.zeros((16,), jnp.float32)

    def quarter(sq, carry):
        h1 = pltpu.async_copy(wd.at[pl.ds(base + sq * 128, 128), :], wdv, semo)
        h2 = pltpu.async_copy(cd.at[pl.ds(base + sq * 128, 128), :], cdv, semo)
        h1.wait()
        h2.wait()

        def unpack(g8, carry2):
            gq = sq * 8 + g8
            rowv = lane + g8 * 16
            sl = pl.ds(gq * 16, 16)

            def wc(j):
                return plsc.load_gather(
                    wdv, [rowv, jnp.full((16,), j, jnp.int32)])

            def cc(j):
                return plsc.load_gather(
                    cdv, [rowv, jnp.full((16,), j, jnp.int32)])

            for j in range(NCTX):
                widx_v[j, sl] = wc(j)
            lens_v[sl] = 0.5 / wc(NCTX).astype(jnp.float32)
            tidx_v[sl] = wc(NCTX + 1)
            for n in range(NEG):
                plsc.store_scatter(nidx_v, [l5 + (gq * 16 * NEG + n)],
                                   wc(NCTX + 2 + n))
                plsc.store_scatter(mask_v, [l5 + (gq * 16 * NEG + n)],
                                   wc(NCTX + 2 + NEG + n))
            for j in range(NCHR):
                cidx_v[j, sl] = cc(j)
            cn_v[sl] = 0.5 / cc(NCHR).astype(jnp.float32)
            return carry2

        lax.fori_loop(0, 8, unpack, 0)
        return carry

    lax.fori_loop(0, 4, quarter, 0)

    def lane_total(v):
        return plsc.cumsum(v).at[idx15].get(mode="promise_in_bounds")

    def zero_bufs(wsum, csum):
        def zrow(i, cz):
            for d in range(4):
                wsum[i, pl.ds(d * 16, 16)] = zero16
                csum[i, pl.ds(d * 16, 16)] = zero16
            return cz
        lax.fori_loop(0, C, zrow, 0)

    def transfers(g, wsum, csum, nrows, trows, sem):
        c0 = g * C
        ts = []
        for j in range(NCTX):
            ts.append((emb0.at[widx_v.at[j, pl.ds(c0, C)]], wsum, sem, True))
        for j in range(NCHR):
            ts.append((emb0c.at[cidx_v.at[j, pl.ds(c0, C)]], csum, sem, True))
        ts.append((emb1.at[nidx_v.at[pl.ds(c0 * NEG, 128)]],
                   nrows.at[pl.ds(0, 128)], sem, False))
        ts.append((emb1.at[nidx_v.at[pl.ds(c0 * NEG + 128, 32)]],
                   nrows.at[pl.ds(128, 32)], sem, False))
        ts.append((emb1.at[tidx_v.at[pl.ds(c0, C)]], trows, sem, False))
        return ts

    def fire(g, wsum, csum, nrows, trows, sem):
        for src, dst, s, add in transfers(g, wsum, csum, nrows, trows, sem):
            pltpu.async_copy(src, dst, s, add=add)

    def drain(g, wsum, csum, nrows, trows, sem):
        for src, dst, s, add in transfers(g, wsum, csum, nrows, trows, sem):
            pltpu.make_async_copy(src, dst, s).wait()

    def compute(g, wsum, csum, nrows, trows):
        c0 = g * C
        for q in range(NQ):
            gq = g * NQ + q
            hlv = lens_v[pl.ds(c0 + q * 16, 16)]
            hcv = cn_v[pl.ds(c0 + q * 16, 16)]

            def row(r2, carry2):
                accs = carry2
                r = q * 16 + r2
                ridx = jnp.full((16,), r2, jnp.int32)
                hl = hlv.at[ridx].get(mode="promise_in_bounds")
                hc = hcv.at[ridx].get(mode="promise_in_bounds")
                avg = []
                for d in range(4):
                    sl = pl.ds(d * 16, 16)
                    avg.append(wsum[r, sl] * hl + csum[r, sl] * hc)
                msk_r = lane == r2
                pv = avg[0] * trows[r, pl.ds(0, 16)]
                for d in range(1, 4):
                    pv = pv + avg[d] * trows[r, pl.ds(d * 16, 16)]
                out = [jnp.where(msk_r, lane_total(pv), accs[0])]
                for n in range(NEG):
                    nv = avg[0] * nrows[r * NEG + n, pl.ds(0, 16)]
                    for d in range(1, 4):
                        nv = nv + avg[d] * nrows[r * NEG + n, pl.ds(d * 16, 16)]
                    out.append(jnp.where(msk_r, lane_total(nv), accs[n + 1]))
                return tuple(out)

            accs = lax.fori_loop(0, 16, row, (zero16,) * (1 + NEG))
            posc[pl.ds(c0 + q * 16, 16)] = jnp.clip(accs[0], -10.0, 10.0)
            for n in range(NEG):
                maskv = plsc.load_gather(mask_v, [l5 + (gq * 16 * NEG + n)])
                sv = jnp.where(maskv == 1,
                               jnp.clip(accs[n + 1], -10.0, 10.0), -90.0)
                plsc.store_scatter(negc, [l5 + ((c0 + q * 16) * NEG + n)], sv)

    bufs0 = (wsum0, csum0, nrows0, trows0, sem0)
    bufs1 = (wsum1, csum1, nrows1, trows1, sem1)

    zero_bufs(wsum0, csum0)
    zero_bufs(wsum1, csum1)
    fire(0, *bufs0)
    fire(1, *bufs1)

    def body(k, carry):
        g0 = 2 * k
        for g, bufs in ((g0, bufs0), (g0 + 1, bufs1)):
            drain(g, *bufs)
            compute(g, *bufs[:4])
            zero_bufs(bufs[0], bufs[1])

            @pl.when(g + 2 < NCHUNK)
            def _():
                fire(g + 2, *bufs)
        return carry

    lax.fori_loop(0, NCHUNK // 2, body, 0)

    o1 = pltpu.async_copy(posc, pos_out.at[pl.ds(base, RPW)], semo)
    o2 = pltpu.async_copy(negc, neg_out.at[pl.ds(base * NEG, RPW * NEG)], semo)
    o1.wait()
    o2.wait()


def _sc_ips(wd, cd, emb0, emb1, emb0c):
    mesh = plsc.VectorSubcoreMesh(core_axis_name="c", subcore_axis_name="s")
    return pl.kernel(
        _sc_body,
        mesh=mesh,
        compiler_params=pltpu.CompilerParams(
            needs_layout_passes=False, use_tc_tiling_on_sc=False),
        out_type=[
            jax.ShapeDtypeStruct((B,), jnp.float32),
            jax.ShapeDtypeStruct((B * NEG,), jnp.float32),
        ],
        scratch_types=[
            pltpu.VMEM((128, 128), jnp.int32),
            pltpu.VMEM((128, 128), jnp.int32),
            pltpu.VMEM((NCTX, RPW), jnp.int32),
            pltpu.VMEM((NCHR, RPW), jnp.int32),
            pltpu.VMEM((RPW * NEG,), jnp.int32),
            pltpu.VMEM((RPW * NEG,), jnp.int32),
            pltpu.VMEM((RPW,), jnp.int32),
            pltpu.VMEM((RPW,), jnp.float32),
            pltpu.VMEM((RPW,), jnp.float32),
            pltpu.VMEM((C, SIZE), jnp.float32),
            pltpu.VMEM((C, SIZE), jnp.float32),
            pltpu.VMEM((C * NEG, SIZE), jnp.float32),
            pltpu.VMEM((C, SIZE), jnp.float32),
            pltpu.VMEM((C, SIZE), jnp.float32),
            pltpu.VMEM((C, SIZE), jnp.float32),
            pltpu.VMEM((C * NEG, SIZE), jnp.float32),
            pltpu.VMEM((C, SIZE), jnp.float32),
            pltpu.VMEM((RPW,), jnp.float32),
            pltpu.VMEM((RPW * NEG,), jnp.float32),
            pltpu.SemaphoreType.DMA,
            pltpu.SemaphoreType.DMA,
            pltpu.SemaphoreType.DMA,
        ],
    )(wd, cd, emb0, emb1, emb0c)


def _tc_loss_body(pos_ref, neg_ref, out_ref):
    pos_loss = jnp.sum(jnp.log(1.0 + jnp.exp(-pos_ref[...])))
    neg_loss = jnp.sum(jnp.log(1.0 + jnp.exp(neg_ref[...])))
    out_ref[0, 0] = pos_loss + neg_loss


def _tc_loss(pos2d, neg2d):
    return pl.pallas_call(
        _tc_loss_body,
        out_shape=jax.ShapeDtypeStruct((1, 1), jnp.float32),
        out_specs=pl.BlockSpec(memory_space=pltpu.SMEM),
    )(pos2d, neg2d)


def kernel(word_data, char_data, emb0_w, emb1_w, emb0_char_w):
    wd = jnp.pad(word_data.astype(jnp.int32), ((0, 0), (0, 128 - WCOL)))
    cd = jnp.pad(char_data.astype(jnp.int32), ((0, 0), (0, 128 - CCOL)))
    pos_ips, neg_ips = _sc_ips(wd, cd, emb0_w, emb1_w, emb0_char_w)
    loss = _tc_loss(pos_ips.reshape(128, 128),
                    neg_ips.reshape(B * NEG // 128, 128))
    return loss[0, 0]


# speedup vs baseline: 1.1269x; 1.0973x over previous
import functools

import jax
import jax.numpy as jnp
from jax import lax
from jax.experimental import pallas as pl
from jax.experimental.pallas import tpu as pltpu
from jax.experimental.pallas import tpu_sc as plsc

B = 16384
WINDOW = 5
NEG = 5
MWL = 5
SIZE = 64
NCTX = 2 * WINDOW
NCHR = 2 * WINDOW * MWL

NC = 2
NS = 16
NW = NC * NS
RPW = B // NW
C = 64
NCHUNK = RPW // C
NQ = C // 16


def _sc_body(ctx_idx, lens, tar, neg, char_idx, cn,
             emb0, emb1, emb0c,
             pos_out, neg_out,
             widx_v, cidx_v, nidx_v, tidx_v, lens_v, cn_v,
             wsum0, csum0, nrows0, trows0,
             wsum1, csum1, nrows1, trows1,
             posc, negc, sem0, sem1):
    wid = lax.axis_index("s") * NC + lax.axis_index("c")
    base = wid * RPW

    pltpu.sync_copy(ctx_idx.at[:, pl.ds(base, RPW)], widx_v)
    pltpu.sync_copy(char_idx.at[:, pl.ds(base, RPW)], cidx_v)
    pltpu.sync_copy(neg.at[pl.ds(base * NEG, RPW * NEG)], nidx_v)
    pltpu.sync_copy(tar.at[pl.ds(base, RPW)], tidx_v)
    pltpu.sync_copy(lens.at[pl.ds(base, RPW)], lens_v)
    pltpu.sync_copy(cn.at[pl.ds(base, RPW)], cn_v)

    lane = lax.broadcasted_iota(jnp.int32, (16,), 0)
    idx15 = jnp.full((16,), 15, jnp.int32)
    zero16 = jnp.zeros((16,), jnp.float32)

    def lane_total(v):
        return plsc.cumsum(v).at[idx15].get(mode="promise_in_bounds")

    def zero_bufs(wsum, csum):
        def zrow(i, cz):
            for d in range(4):
                wsum[i, pl.ds(d * 16, 16)] = zero16
                csum[i, pl.ds(d * 16, 16)] = zero16
            return cz
        lax.fori_loop(0, C, zrow, 0)

    def transfers(g, wsum, csum, nrows, trows, sem):
        c0 = g * C
        ts = []
        for j in range(NCTX):
            ts.append((emb0.at[widx_v.at[j, pl.ds(c0, C)]], wsum, sem, True))
        for j in range(NCHR):
            ts.append((emb0c.at[cidx_v.at[j, pl.ds(c0, C)]], csum, sem, True))
        for t in range(2):
            ts.append((emb1.at[nidx_v.at[pl.ds(c0 * NEG + t * 128, 128)]],
                       nrows.at[pl.ds(t * 128, 128)], sem, False))
        ts.append((emb1.at[nidx_v.at[pl.ds(c0 * NEG + 256, 64)]],
                   nrows.at[pl.ds(256, 64)], sem, False))
        ts.append((emb1.at[tidx_v.at[pl.ds(c0, C)]], trows, sem, False))
        return ts

    def fire(g, wsum, csum, nrows, trows, sem):
        for src, dst, s, add in transfers(g, wsum, csum, nrows, trows, sem):
            pltpu.async_copy(src, dst, s, add=add)

    def drain(g, wsum, csum, nrows, trows, sem):
        for src, dst, s, add in transfers(g, wsum, csum, nrows, trows, sem):
            pltpu.make_async_copy(src, dst, s).wait()

    def compute(g, wsum, csum, nrows, trows):
        c0 = g * C
        for q in range(NQ):
            hlv = 0.5 / lens_v[pl.ds(c0 + q * 16, 16)]
            hcv = 0.5 / cn_v[pl.ds(c0 + q * 16, 16)]

            def row(r2, carry2):
                accs = carry2
                r = q * 16 + r2
                ridx = jnp.full((16,), r2, jnp.int32)
                hl = hlv.at[ridx].get(mode="promise_in_bounds")
                hc = hcv.at[ridx].get(mode="promise_in_bounds")
                avg = []
                for d in range(4):
                    sl = pl.ds(d * 16, 16)
                    avg.append(wsum[r, sl] * hl + csum[r, sl] * hc)
                msk_r = lane == r2
                pv = avg[0] * trows[r, pl.ds(0, 16)]
                for d in range(1, 4):
                    pv = pv + avg[d] * trows[r, pl.ds(d * 16, 16)]
                out = [jnp.where(msk_r, lane_total(pv), accs[0])]
                for n in range(NEG):
                    nv = avg[0] * nrows[r * NEG + n, pl.ds(0, 16)]
                    for d in range(1, 4):
                        nv = nv + avg[d] * nrows[r * NEG + n, pl.ds(d * 16, 16)]
                    out.append(jnp.where(msk_r, lane_total(nv), accs[n + 1]))
                return tuple(out)

            accs = lax.fori_loop(0, 16, row, (zero16,) * (1 + NEG))
            posc[pl.ds(q * 16, 16)] = accs[0]
            for n in range(NEG):
                negc[n, pl.ds(q * 16, 16)] = accs[n + 1]

        pltpu.sync_copy(posc, pos_out.at[pl.ds(base + c0, C)])
        for n in range(NEG):
            pltpu.sync_copy(negc.at[n], neg_out.at[pl.ds(n * B + base + c0, C)])

    bufs0 = (wsum0, csum0, nrows0, trows0, sem0)
    bufs1 = (wsum1, csum1, nrows1, trows1, sem1)

    zero_bufs(wsum0, csum0)
    zero_bufs(wsum1, csum1)
    fire(0, *bufs0)
    fire(1, *bufs1)

    def body(k, carry):
        g0 = 2 * k
        for g, bufs in ((g0, bufs0), (g0 + 1, bufs1)):
            drain(g, *bufs)
            compute(g, *bufs[:4])
            zero_bufs(bufs[0], bufs[1])

            @pl.when(g + 2 < NCHUNK)
            def _():
                fire(g + 2, *bufs)
        return carry

    lax.fori_loop(0, NCHUNK // 2, body, 0)


def _sc_ips(ctx_idx, lens, tar, neg, char_idx, cn, emb0, emb1, emb0c):
    mesh = plsc.VectorSubcoreMesh(core_axis_name="c", subcore_axis_name="s")
    return pl.kernel(
        _sc_body,
        mesh=mesh,
        compiler_params=pltpu.CompilerParams(
            needs_layout_passes=False, use_tc_tiling_on_sc=False),
        out_type=[
            jax.ShapeDtypeStruct((B,), jnp.float32),
            jax.ShapeDtypeStruct((B * NEG,), jnp.float32),
        ],
        scratch_types=[
            pltpu.VMEM((NCTX, RPW), jnp.int32),
            pltpu.VMEM((NCHR, RPW), jnp.int32),
            pltpu.VMEM((RPW * NEG,), jnp.int32),
            pltpu.VMEM((RPW,), jnp.int32),
            pltpu.VMEM((RPW,), jnp.float32),
            pltpu.VMEM((RPW,), jnp.float32),
            pltpu.VMEM((C, SIZE), jnp.float32),
            pltpu.VMEM((C, SIZE), jnp.float32),
            pltpu.VMEM((C * NEG, SIZE), jnp.float32),
            pltpu.VMEM((C, SIZE), jnp.float32),
            pltpu.VMEM((C, SIZE), jnp.float32),
            pltpu.VMEM((C, SIZE), jnp.float32),
            pltpu.VMEM((C * NEG, SIZE), jnp.float32),
            pltpu.VMEM((C, SIZE), jnp.float32),
            pltpu.VMEM((C,), jnp.float32),
            pltpu.VMEM((NEG, C), jnp.float32),
            pltpu.SemaphoreType.DMA,
            pltpu.SemaphoreType.DMA,
        ],
    )(ctx_idx, lens, tar, neg, char_idx, cn, emb0, emb1, emb0c)


def _tc_loss_body(pos_ref, neg_ref, mask_ref, out_ref):
    p = jnp.clip(pos_ref[...], -10.0, 10.0)
    pos_loss = jnp.sum(jnp.log(1.0 + jnp.exp(-p)))
    m = mask_ref[...]
    q = jnp.clip(neg_ref[...] * m, -10.0, 10.0)
    neg_loss = jnp.sum(jnp.log(1.0 + jnp.exp(q)) * m)
    out_ref[0, 0] = pos_loss + neg_loss


def _tc_loss(pos2d, neg2d, mask2d):
    return pl.pallas_call(
        _tc_loss_body,
        out_shape=jax.ShapeDtypeStruct((1, 1), jnp.float32),
        out_specs=pl.BlockSpec(memory_space=pltpu.SMEM),
    )(pos2d, neg2d, mask2d)


def kernel(word_data, char_data, emb0_w, emb1_w, emb0_char_w):
    wd = word_data.astype(jnp.int32)
    cd = char_data.astype(jnp.int32)
    ctx_idx = wd[:, 0:NCTX].T
    lens = wd[:, NCTX].astype(jnp.float32)
    tar = wd[:, NCTX + 1]
    neg = wd[:, NCTX + 2:NCTX + 2 + NEG].reshape(-1)
    mask = wd[:, NCTX + 2 + NEG:].astype(jnp.float32)
    char_idx = cd[:, 0:NCHR].T
    cn = cd[:, NCHR].astype(jnp.float32)

    pos_ips, neg_ips = _sc_ips(ctx_idx, lens, tar, neg, char_idx, cn,
                               emb0_w, emb1_w, emb0_char_w)
    loss = _tc_loss(pos_ips.reshape(128, 128),
                    neg_ips.reshape(B * NEG // 128, 128),
                    mask.T.reshape(B * NEG // 128, 128))
    return loss[0, 0]
